# baseline (speedup 1.0000x reference)
import functools

import jax
import jax.numpy as jnp
from jax import lax
from jax.experimental import pallas as pl
from jax.experimental.pallas import tpu as pltpu
from jax.experimental.pallas import tpu_sc as plsc

BATCH = 4096
SEQ = 200
DIM = 64
PAD = 128
NUM_WORKERS = 32
CHUNK = 128
NB = BATCH // CHUNK
N_PAIRS = SEQ // 2


def _make_gather():
    mesh = plsc.VectorSubcoreMesh(core_axis_name="c", subcore_axis_name="s")

    @functools.partial(
        pl.kernel,
        mesh=mesh,
        out_type=jax.ShapeDtypeStruct((SEQ, 8, NB, 8, CHUNK), jnp.float32),
        scratch_types=[
            pltpu.VMEM((SEQ, CHUNK), jnp.int32),
            pltpu.VMEM((CHUNK, PAD), jnp.float32),
            pltpu.VMEM((CHUNK, PAD), jnp.float32),
            pltpu.VMEM((CHUNK, PAD), jnp.float32),
            pltpu.VMEM((CHUNK, PAD), jnp.float32),
            pltpu.VMEM((8, 8, CHUNK), jnp.float32),
            pltpu.VMEM((8, 8, CHUNK), jnp.float32),
            pltpu.SemaphoreType.DMA,
            pltpu.SemaphoreType.DMA,
            pltpu.SemaphoreType.DMA,
            pltpu.SemaphoreType.DMA,
            pltpu.SemaphoreType.DMA,
            pltpu.SemaphoreType.DMA,
        ],
        compiler_params=pltpu.CompilerParams(needs_layout_passes=False),
    )
    def gather_kernel(idx_hbm, table_hbm, out_hbm, idx_v, r0, r1, r2, r3,
                      t0, t1, g0, g1, g2, g3, o0, o1):
        w = lax.axis_index("s") * 2 + lax.axis_index("c")
        pltpu.sync_copy(idx_hbm.at[:, w], idx_v)

        rbufs = [r0, r1, r2, r3]
        gsems = [g0, g1, g2, g3]
        tbufs = [t0, t1]
        osems = [o0, o1]

        lane = lax.iota(jnp.int32, 16)
        c_idx = [lane + 16 * gi for gi in range(8)]

        def fire(s, rbuf, sem):
            pltpu.async_copy(table_hbm.at[idx_v.at[s]], rbuf, sem)

        def gather_wait(rbuf, sem):
            pltpu.make_async_copy(table_hbm.at[idx_v.at[0]], rbuf, sem).wait()

        def transpose(rbuf, tbuf):
            @plsc.parallel_loop(0, DIM, unroll=4)
            def e_body(e):
                te = lax.div(e, 8)
                r = lax.rem(e, 8)
                e_idx = jnp.zeros((16,), jnp.int32) + e
                for gi in range(8):
                    tbuf[te, r, pl.ds(16 * gi, 16)] = plsc.load_gather(
                        rbuf, [c_idx[gi], e_idx])

        def out_start(s, tbuf, sem):
            pltpu.async_copy(tbuf, out_hbm.at[s, :, w], sem)

        def out_wait(tbuf, sem):
            pltpu.make_async_copy(tbuf, out_hbm.at[0, :, w], sem).wait()

        def step(s_proc, s_fire, k, first):
            gather_wait(rbufs[k], gsems[k])
            if not first:
                out_wait(tbufs[k % 2], osems[k % 2])
            transpose(rbufs[k], tbufs[k % 2])
            out_start(s_proc, tbufs[k % 2], osems[k % 2])
            fire(s_fire, rbufs[k], gsems[k])

        for k in range(4):
            fire(k, rbufs[k], gsems[k])
        for k in range(4):
            step(k, k + 4, k, k < 2)

        def body(p, carry):
            s0 = 4 * p
            for k in range(4):
                step(s0 + k, jnp.minimum(s0 + k + 4, SEQ - 1), k, False)
            return carry

        lax.fori_loop(1, SEQ // 4, body, 0)
        for k in range(4):
            gather_wait(rbufs[k], gsems[k])
        out_wait(t0, o0)
        out_wait(t1, o1)

    return gather_kernel


_gather = _make_gather()


def kernel(word_inputs, word_seq_lengths, char_inputs, char_seq_lengths,
           char_seq_recover, word_embeddings):
    idx = word_inputs.T.astype(jnp.int32).reshape(SEQ, NB, CHUNK)
    table = jnp.pad(word_embeddings, ((0, 0), (0, PAD - DIM)))
    x = _gather(idx, table)
    return x.transpose(2, 4, 0, 1, 3).reshape(BATCH, SEQ, DIM)
